# baseline (speedup 1.0000x reference)
import jax
import jax.numpy as jnp
from jax import lax
from jax.experimental import pallas as pl
from jax.experimental.pallas import tpu as pltpu


def _round_up(x: int, m: int) -> int:
    return ((x + m - 1) // m) * m


def _fused_kernel(u_ref, w_ref, b_ref, item_ref, uf_ref, res_ref):
    uf = lax.dot_general(
        u_ref[...], w_ref[...],
        dimension_numbers=(((1,), (1,)), ((), ())),
        preferred_element_type=jnp.float32,
    ) + b_ref[...]
    uf_ref[...] = uf.astype(uf_ref.dtype)
    res = lax.dot_general(
        uf, item_ref[...],
        dimension_numbers=(((1,), (1,)), ((), ())),
        preferred_element_type=jnp.float32,
    )
    res_ref[...] = res.astype(res_ref.dtype)


@jax.jit
def _forward(user_feat, item_feat, weight, bias):
    U, F = user_feat.shape
    I, _ = item_feat.shape
    isz = jnp.dtype(user_feat.dtype).itemsize

    tu = min(256, _round_up(U, 8))
    U_pad = _round_up(U, tu)

    user_p = user_feat if U_pad == U else jnp.pad(user_feat, ((0, U_pad - U), (0, 0)))
    bias2d = bias.reshape(1, F)

    ufeat_p, result = pl.pallas_call(
        _fused_kernel,
        out_shape=(
            jax.ShapeDtypeStruct((U_pad, F), user_feat.dtype),
            jax.ShapeDtypeStruct((U_pad, I), user_feat.dtype),
        ),
        grid=(U_pad // tu,),
        in_specs=[
            pl.BlockSpec((tu, F), lambda i: (i, 0)),
            pl.BlockSpec((F, F), lambda i: (0, 0)),
            pl.BlockSpec((1, F), lambda i: (0, 0)),
            pl.BlockSpec((I, F), lambda i: (0, 0)),
        ],
        out_specs=(
            pl.BlockSpec((tu, F), lambda i: (i, 0)),
            pl.BlockSpec((tu, I), lambda i: (i, 0)),
        ),
        compiler_params=pltpu.CompilerParams(
            dimension_semantics=("parallel",),
            vmem_limit_bytes=110 * 1024 * 1024,
        ),
        cost_estimate=pl.CostEstimate(
            flops=2 * U_pad * F * (F + I),
            transcendentals=0,
            bytes_accessed=isz * (U_pad * F * 2 + F * F + F + I * F + U_pad * I),
        ),
    )(user_p, weight, bias2d, item_feat)

    ufeat = ufeat_p if U_pad == U else ufeat_p[:U, :]
    result = result if U_pad == U else result[:U, :]
    return ufeat, result


def kernel(user_feat, item_feat, weight, bias):
    return _forward(user_feat, item_feat, weight, bias)


# speedup vs baseline: 1.0557x; 1.0557x over previous
import jax
import jax.numpy as jnp
from jax import lax
from jax.experimental import pallas as pl
from jax.experimental.pallas import tpu as pltpu


def _round_up(x: int, m: int) -> int:
    return ((x + m - 1) // m) * m


def _fused_kernel(u_ref, w_ref, b_ref, item_ref, uf_ref, res_ref):
    uf = lax.dot_general(
        u_ref[...], w_ref[...],
        dimension_numbers=(((1,), (1,)), ((), ())),
        preferred_element_type=jnp.float32,
    ) + b_ref[...]
    uf_ref[...] = uf.astype(uf_ref.dtype)
    res = lax.dot_general(
        uf, item_ref[...],
        dimension_numbers=(((1,), (1,)), ((), ())),
        preferred_element_type=jnp.float32,
    )
    res_ref[...] = res.astype(res_ref.dtype)


@jax.jit
def _forward(user_feat, item_feat, weight, bias):
    U, F = user_feat.shape
    I, _ = item_feat.shape
    isz = jnp.dtype(user_feat.dtype).itemsize

    tu = min(512, _round_up(U, 8))
    U_pad = _round_up(U, tu)

    user_p = user_feat if U_pad == U else jnp.pad(user_feat, ((0, U_pad - U), (0, 0)))
    bias2d = bias.reshape(1, F)

    ufeat_p, result = pl.pallas_call(
        _fused_kernel,
        out_shape=(
            jax.ShapeDtypeStruct((U_pad, F), user_feat.dtype),
            jax.ShapeDtypeStruct((U_pad, I), user_feat.dtype),
        ),
        grid=(U_pad // tu,),
        in_specs=[
            pl.BlockSpec((tu, F), lambda i: (i, 0)),
            pl.BlockSpec((F, F), lambda i: (0, 0)),
            pl.BlockSpec((1, F), lambda i: (0, 0)),
            pl.BlockSpec((I, F), lambda i: (0, 0)),
        ],
        out_specs=(
            pl.BlockSpec((tu, F), lambda i: (i, 0)),
            pl.BlockSpec((tu, I), lambda i: (i, 0)),
        ),
        compiler_params=pltpu.CompilerParams(
            dimension_semantics=("parallel",),
            vmem_limit_bytes=110 * 1024 * 1024,
        ),
        cost_estimate=pl.CostEstimate(
            flops=2 * U_pad * F * (F + I),
            transcendentals=0,
            bytes_accessed=isz * (U_pad * F * 2 + F * F + F + I * F + U_pad * I),
        ),
    )(user_p, weight, bias2d, item_feat)

    ufeat = ufeat_p if U_pad == U else ufeat_p[:U, :]
    result = result if U_pad == U else result[:U, :]
    return ufeat, result


def kernel(user_feat, item_feat, weight, bias):
    return _forward(user_feat, item_feat, weight, bias)
